# baseline (speedup 1.0000x reference)
import functools

import jax
import jax.numpy as jnp
import numpy as np
from jax import lax
from jax.experimental import pallas as pl
from jax.experimental.pallas import tpu as pltpu
from jax.experimental.pallas import tpu_sc as plsc

_N = 100000
_D1 = 768
_D2 = 1024
_TOPM = 1000
_TOPK = 10

_BLK = 2048
_G1 = 49
_NPAD = _G1 * _BLK
_MININT = np.int32(-2147483648)
_SENTINEL = np.int32(2 ** 30)

_HALF = 1280
_SLOTS = 2 * _HALF


def _sortable(bits):
    return jnp.where(
        bits < 0,
        jnp.bitwise_xor(jnp.bitwise_not(bits), _MININT),
        bits,
    )


def _stage1_body(x_ref, t_ref, s_ref, tj_ref, k_ref):
    i = pl.program_id(0)
    x = x_ref[...]
    t = t_ref[...]
    tn = t / jnp.maximum(jnp.sqrt(jnp.sum(t * t)), 1e-8)
    sq = jnp.sum(x * x, axis=1, keepdims=True)
    xn = x / jnp.maximum(jnp.sqrt(sq), 1e-8)
    tb = jnp.broadcast_to(tn, (8, _D1)).astype(jnp.bfloat16)
    score = lax.dot_general(xn.astype(jnp.bfloat16), tb,
                            (((1,), (1,)), ((), ())),
                            preferred_element_type=jnp.float32)[:, 0:1]
    gidx = i * _BLK + lax.broadcasted_iota(jnp.int32, (_BLK, 1), 0)
    score = jnp.where(gidx < _N, score, -jnp.inf)
    s_ref[pl.ds(i, 1), :] = score.reshape(1, _BLK)

    @pl.when(i == _G1 - 1)
    def _select():
        bits = lax.bitcast_convert_type(s_ref[...], jnp.int32)
        k_ref[...] = _sortable(bits)

        def t_body(b, t_u):
            cand = jnp.bitwise_or(t_u, lax.shift_left(jnp.int32(1), 31 - b))
            cnt = jnp.sum((k_ref[...] >=
                           jnp.bitwise_xor(cand, _MININT)).astype(jnp.int32))
            return jnp.where(cnt >= _TOPM, cand, t_u)

        t_u = lax.fori_loop(0, 32, t_body, jnp.int32(0))
        t_k = jnp.bitwise_xor(t_u, _MININT)

        flat = (lax.broadcasted_iota(jnp.int32, (_G1, _BLK), 0) * _BLK
                + lax.broadcasted_iota(jnp.int32, (_G1, _BLK), 1))
        n_gt = jnp.sum((k_ref[...] > t_k).astype(jnp.int32))

        def g_of(j):
            eq = k_ref[...] == t_k
            return n_gt + jnp.sum((eq & (flat <= j)).astype(jnp.int32))

        def j_body(b, j_bad):
            cand = jnp.bitwise_or(j_bad, lax.shift_left(jnp.int32(1), 16 - b))
            return jnp.where(g_of(cand) < _TOPM, cand, j_bad)

        j_bad = lax.fori_loop(0, 17, j_body, jnp.int32(0))
        j_fin = jnp.where(g_of(jnp.int32(0)) >= _TOPM, jnp.int32(0), j_bad + 1)

        t_bits = jnp.where(
            t_k >= 0, t_k, jnp.bitwise_not(jnp.bitwise_xor(t_k, _MININT)))
        r = lax.broadcasted_iota(jnp.int32, (8, 128), 0)
        tj_ref[...] = jnp.where(r == 0, t_bits, jnp.where(r == 1, j_fin, 0))


def _stage1(images_emb1, text_emb1):
    return pl.pallas_call(
        _stage1_body,
        grid=(_G1,),
        in_specs=[
            pl.BlockSpec((_BLK, _D1), lambda i: (i, 0)),
            pl.BlockSpec((1, _D1), lambda i: (0, 0)),
        ],
        out_specs=[
            pl.BlockSpec((_G1, _BLK), lambda i: (0, 0)),
            pl.BlockSpec((8, 128), lambda i: (0, 0)),
        ],
        out_shape=[
            jax.ShapeDtypeStruct((_G1, _BLK), jnp.float32),
            jax.ShapeDtypeStruct((8, 128), jnp.int32),
        ],
        scratch_shapes=[pltpu.VMEM((_G1, _BLK), jnp.int32)],
    )(images_emb1, text_emb1.reshape(1, _D1))


_PER_TILE = _NPAD // 32
_CHUNKS = _PER_TILE // 16
_ROWS_PER_TILE = _SLOTS // 32


def _make_compact():
    mesh = plsc.VectorSubcoreMesh(core_axis_name="c", subcore_axis_name="s")

    @functools.partial(
        pl.kernel,
        mesh=mesh,
        compiler_params=pltpu.CompilerParams(needs_layout_passes=False),
        out_type=[
            jax.ShapeDtypeStruct((_SLOTS,), jnp.int32),
            jax.ShapeDtypeStruct((512,), jnp.int32),
            jax.ShapeDtypeStruct((_SLOTS, _D2), jnp.float32),
        ],
        scratch_types=[
            pltpu.VMEM((_PER_TILE,), jnp.float32),
            pltpu.VMEM((16,), jnp.float32),
            pltpu.VMEM((16,), jnp.int32),
            pltpu.VMEM((_PER_TILE + 64,), jnp.int32),
            pltpu.VMEM((16,), jnp.int32),
            pltpu.VMEM((256,), jnp.int32),
            pltpu.VMEM((_ROWS_PER_TILE,), jnp.int32),
            pltpu.VMEM((_ROWS_PER_TILE, _D2), jnp.float32),
            pltpu.SemaphoreType.DMA,
        ],
    )
    def compact(s_hbm, img2_hbm, t_hbm, j_hbm, cand_hbm, cnt_hbm, rows_hbm,
                sv, tv, jv, locbuf, cv, rb, idxv, rowsb, sem):
        cid = lax.axis_index("c")
        sid = lax.axis_index("s")
        tile = cid * 16 + sid
        base = tile * _PER_TILE
        pltpu.sync_copy(s_hbm.at[pl.ds(pl.multiple_of(base, 8), _PER_TILE)], sv)
        pltpu.sync_copy(t_hbm, tv)
        pltpu.sync_copy(j_hbm, jv)
        tvv = tv[...]
        jvv = jv[...]

        def body(c, cnt):
            off = c * 16
            s = sv[pl.ds(off, 16)]
            gidx = (base + off) + lax.iota(jnp.int32, 16)
            sel = (s > tvv) | ((s == tvv) & (gidx <= jvv))
            seli = sel.astype(jnp.int32)
            pos = cnt + plsc.cumsum(seli) - 1
            plsc.store_scatter(locbuf, [pos], gidx, mask=sel)
            return cnt + jnp.sum(seli)

        cnt = lax.fori_loop(0, _CHUNKS, body, jnp.int32(0))

        npad = lax.rem(jnp.int32(16) - lax.rem(cnt, jnp.int32(16)),
                       jnp.int32(16))
        padmask = lax.iota(jnp.int32, 16) < npad
        plsc.store_scatter(locbuf, [cnt + lax.iota(jnp.int32, 16)],
                           jnp.full((16,), _SENTINEL, jnp.int32), mask=padmask)
        cntp = cnt + npad

        cv[...] = jnp.zeros((16,), jnp.int32) + cntp
        pltpu.sync_copy(cv, cnt_hbm.at[pl.ds(pl.multiple_of(tile * 16, 8), 16)])
        plsc.subcore_barrier()
        pltpu.sync_copy(cnt_hbm.at[pl.ds(pl.multiple_of(cid * 256, 8), 256)], rb)
        off0 = jnp.int32(0)
        tot_c = jnp.int32(0)
        for t in range(16):
            c_t = jnp.max(rb[pl.ds(t * 16, 16)])
            off0 = off0 + jnp.where(sid > t, c_t, 0)
            tot_c = tot_c + c_t
        dest = cid * _HALF + off0

        def wr(k, carry):
            pltpu.sync_copy(locbuf.at[pl.ds(k * 16, 16)],
                            cand_hbm.at[pl.ds(pl.multiple_of(dest + k * 16, 8), 16)])
            return carry

        lax.fori_loop(0, cntp // 16, wr, jnp.int32(0))

        plsc.subcore_barrier()
        for j in range(_HALF // 256):
            gstart = (sid + j * 16) * 16

            @pl.when(gstart < tot_c)
            def _gather(j=j, gstart=gstart):
                gslot = cid * _HALF + gstart
                pltpu.sync_copy(
                    cand_hbm.at[pl.ds(pl.multiple_of(gslot, 8), 16)],
                    idxv.at[pl.ds(j * 16, 16)])
                v = idxv[pl.ds(j * 16, 16)]
                idxv[pl.ds(j * 16, 16)] = jnp.clip(v, 0, _N - 1)
                pltpu.async_copy(
                    img2_hbm.at[idxv.at[pl.ds(j * 16, 16)]],
                    rowsb.at[pl.ds(j * 16, 16)], sem).wait()
                pltpu.sync_copy(
                    rowsb.at[pl.ds(j * 16, 16)],
                    rows_hbm.at[pl.ds(pl.multiple_of(gslot, 8), 16)])

    return compact


_RBLK = 512
_G2 = _SLOTS // _RBLK


def _score2_body(c_ref, r_ref, cnt_ref, t_ref, oc_ref, ov_ref, s_ref):
    i = pl.program_id(0)
    rows = r_ref[...]
    t2 = t_ref[...]
    tn = t2 / jnp.maximum(jnp.sqrt(jnp.sum(t2 * t2)), 1e-8)
    sq = jnp.sum(rows * rows, axis=1, keepdims=True)
    en = rows / jnp.maximum(jnp.sqrt(sq), 1e-8)
    tb = jnp.broadcast_to(tn, (8, _D2)).astype(jnp.bfloat16)
    s = lax.dot_general(en.astype(jnp.bfloat16), tb,
                        (((1,), (1,)), ((), ())),
                        preferred_element_type=jnp.float32)[:, 0:1]
    s_ref[pl.ds(i * (_RBLK // 128), _RBLK // 128), :] = s.reshape(
        _RBLK // 128, 128)

    @pl.when(i == _G2 - 1)
    def _top():
        cand = c_ref[...]
        counts = cnt_ref[...]
        s2 = s_ref[...]

        rows32 = lax.broadcasted_iota(jnp.int32, (32, 16), 0)
        lane16 = lax.broadcasted_iota(jnp.int32, (32, 16), 1)
        cnt_lane0 = jnp.where(lane16 == 0, counts, 0)
        tot0 = jnp.sum(jnp.where(rows32 < 16, cnt_lane0, 0))
        tot1 = jnp.sum(jnp.where(rows32 >= 16, cnt_lane0, 0))

        flat = (lax.broadcasted_iota(jnp.int32, (20, 128), 0) * 128
                + lax.broadcasted_iota(jnp.int32, (20, 128), 1))
        s_in = lax.rem(flat, jnp.int32(_HALF))
        tot = jnp.where(flat < _HALF, tot0, tot1)
        valid = (s_in < tot) & (cand >= 0) & (cand < _N)
        s2 = jnp.where(valid, s2, -jnp.inf)

        big = jnp.int32(2 ** 30)
        out_r = lax.broadcasted_iota(jnp.int32, (8, 128), 0)
        out_l = lax.broadcasted_iota(jnp.int32, (8, 128), 1)
        oc = jnp.zeros((8, 128), jnp.int32)
        ov = jnp.zeros((8, 128), jnp.float32)
        for k in range(_TOPK):
            mx = jnp.max(s2)
            hit = s2 == mx
            fidx = jnp.min(jnp.where(hit, flat, big))
            sel = flat == fidx
            cbest = jnp.sum(jnp.where(sel, cand, 0))
            oc = jnp.where((out_r == 0) & (out_l == k), cbest, oc)
            ov = jnp.where((out_r == 0) & (out_l == k), mx, ov)
            s2 = jnp.where(sel, -jnp.inf, s2)
        oc_ref[...] = oc
        ov_ref[...] = ov


def _top10(cand, rows, counts, text_emb2):
    return pl.pallas_call(
        _score2_body,
        grid=(_G2,),
        in_specs=[
            pl.BlockSpec((20, 128), lambda i: (0, 0)),
            pl.BlockSpec((_RBLK, _D2), lambda i: (i, 0)),
            pl.BlockSpec((32, 16), lambda i: (0, 0)),
            pl.BlockSpec((1, _D2), lambda i: (0, 0)),
        ],
        out_specs=[
            pl.BlockSpec((8, 128), lambda i: (0, 0)),
            pl.BlockSpec((8, 128), lambda i: (0, 0)),
        ],
        out_shape=[
            jax.ShapeDtypeStruct((8, 128), jnp.int32),
            jax.ShapeDtypeStruct((8, 128), jnp.float32),
        ],
        scratch_shapes=[pltpu.VMEM((20, 128), jnp.float32)],
    )(
        cand.reshape(20, 128),
        rows,
        counts.reshape(32, 16),
        text_emb2.reshape(1, _D2),
    )


def kernel(images_emb1, text_emb1, images_emb2, text_emb2, topm, topk):
    scores2d, tj = _stage1(images_emb1, text_emb1)

    tvec = lax.bitcast_convert_type(jnp.broadcast_to(tj[0:1, 0], (16,)),
                                    jnp.float32)
    jvec = jnp.broadcast_to(tj[1:2, 0], (16,))
    cand, counts, rows = _make_compact()(
        scores2d.reshape(_NPAD), images_emb2, tvec, jvec)

    oc, ov = _top10(cand, rows, counts, text_emb2)
    final_candidate = oc[0, :_TOPK]
    top_vals = ov[0, :_TOPK]
    return final_candidate, top_vals


# speedup vs baseline: 1.0724x; 1.0724x over previous
import functools

import jax
import jax.numpy as jnp
import numpy as np
from jax import lax
from jax.experimental import pallas as pl
from jax.experimental.pallas import tpu as pltpu
from jax.experimental.pallas import tpu_sc as plsc

_N = 100000
_D1 = 768
_D2 = 1024
_TOPM = 1000
_TOPK = 10

_BLK = 4096
_G1 = 25
_NPAD = _G1 * _BLK
_MININT = np.int32(-2147483648)
_SENTINEL = np.int32(2 ** 30)

_HALF = 1280
_SLOTS = 2 * _HALF


def _sortable(bits):
    return jnp.where(
        bits < 0,
        jnp.bitwise_xor(jnp.bitwise_not(bits), _MININT),
        bits,
    )


def _stage1_body(x_ref, t_ref, s_ref, tj_ref, k_ref):
    i = pl.program_id(0)
    x = x_ref[...]
    t = t_ref[...]
    tn = t / jnp.maximum(jnp.sqrt(jnp.sum(t * t)), 1e-8)
    sq = jnp.sum(x * x, axis=1, keepdims=True)
    xn = x / jnp.maximum(jnp.sqrt(sq), 1e-8)
    tb = jnp.broadcast_to(tn, (8, _D1)).astype(jnp.bfloat16)
    score = lax.dot_general(xn.astype(jnp.bfloat16), tb,
                            (((1,), (1,)), ((), ())),
                            preferred_element_type=jnp.float32)[:, 0:1]
    gidx = i * _BLK + lax.broadcasted_iota(jnp.int32, (_BLK, 1), 0)
    score = jnp.where(gidx < _N, score, -jnp.inf)
    s_ref[pl.ds(i, 1), :] = score.reshape(1, _BLK)

    @pl.when(i == _G1 - 1)
    def _select():
        bits = lax.bitcast_convert_type(s_ref[...], jnp.int32)
        k_ref[...] = _sortable(bits)

        def t_body(b, t_u):
            cand = jnp.bitwise_or(t_u, lax.shift_left(jnp.int32(1), 31 - b))
            cnt = jnp.sum((k_ref[...] >=
                           jnp.bitwise_xor(cand, _MININT)).astype(jnp.int32))
            return jnp.where(cnt >= _TOPM, cand, t_u)

        t_u = lax.fori_loop(0, 32, t_body, jnp.int32(0))
        t_k = jnp.bitwise_xor(t_u, _MININT)

        flat = (lax.broadcasted_iota(jnp.int32, (_G1, _BLK), 0) * _BLK
                + lax.broadcasted_iota(jnp.int32, (_G1, _BLK), 1))
        n_gt = jnp.sum((k_ref[...] > t_k).astype(jnp.int32))

        def g_of(j):
            eq = k_ref[...] == t_k
            return n_gt + jnp.sum((eq & (flat <= j)).astype(jnp.int32))

        def j_body(b, j_bad):
            cand = jnp.bitwise_or(j_bad, lax.shift_left(jnp.int32(1), 16 - b))
            return jnp.where(g_of(cand) < _TOPM, cand, j_bad)

        j_bad = lax.fori_loop(0, 17, j_body, jnp.int32(0))
        j_fin = jnp.where(g_of(jnp.int32(0)) >= _TOPM, jnp.int32(0), j_bad + 1)

        t_bits = jnp.where(
            t_k >= 0, t_k, jnp.bitwise_not(jnp.bitwise_xor(t_k, _MININT)))
        r = lax.broadcasted_iota(jnp.int32, (8, 128), 0)
        tj_ref[...] = jnp.where(r == 0, t_bits, jnp.where(r == 1, j_fin, 0))


def _stage1(images_emb1, text_emb1):
    return pl.pallas_call(
        _stage1_body,
        grid=(_G1,),
        in_specs=[
            pl.BlockSpec((_BLK, _D1), lambda i: (i, 0)),
            pl.BlockSpec((1, _D1), lambda i: (0, 0)),
        ],
        out_specs=[
            pl.BlockSpec((_G1, _BLK), lambda i: (0, 0)),
            pl.BlockSpec((8, 128), lambda i: (0, 0)),
        ],
        out_shape=[
            jax.ShapeDtypeStruct((_G1, _BLK), jnp.float32),
            jax.ShapeDtypeStruct((8, 128), jnp.int32),
        ],
        scratch_shapes=[pltpu.VMEM((_G1, _BLK), jnp.int32)],
    )(images_emb1, text_emb1.reshape(1, _D1))


_PER_TILE = _NPAD // 32
_CHUNKS = _PER_TILE // 16
_ROWS_PER_TILE = _SLOTS // 32


def _make_compact():
    mesh = plsc.VectorSubcoreMesh(core_axis_name="c", subcore_axis_name="s")

    @functools.partial(
        pl.kernel,
        mesh=mesh,
        compiler_params=pltpu.CompilerParams(needs_layout_passes=False),
        out_type=[
            jax.ShapeDtypeStruct((_SLOTS,), jnp.int32),
            jax.ShapeDtypeStruct((512,), jnp.int32),
            jax.ShapeDtypeStruct((_SLOTS, _D2), jnp.float32),
        ],
        scratch_types=[
            pltpu.VMEM((_PER_TILE,), jnp.float32),
            pltpu.VMEM((16,), jnp.float32),
            pltpu.VMEM((16,), jnp.int32),
            pltpu.VMEM((_PER_TILE + 64,), jnp.int32),
            pltpu.VMEM((16,), jnp.int32),
            pltpu.VMEM((256,), jnp.int32),
            pltpu.VMEM((_ROWS_PER_TILE,), jnp.int32),
            pltpu.VMEM((_ROWS_PER_TILE, _D2), jnp.float32),
            pltpu.SemaphoreType.DMA,
        ],
    )
    def compact(s_hbm, img2_hbm, t_hbm, j_hbm, cand_hbm, cnt_hbm, rows_hbm,
                sv, tv, jv, locbuf, cv, rb, idxv, rowsb, sem):
        cid = lax.axis_index("c")
        sid = lax.axis_index("s")
        tile = cid * 16 + sid
        base = tile * _PER_TILE
        pltpu.sync_copy(s_hbm.at[pl.ds(pl.multiple_of(base, 8), _PER_TILE)], sv)
        pltpu.sync_copy(t_hbm, tv)
        pltpu.sync_copy(j_hbm, jv)
        tvv = tv[...]
        jvv = jv[...]

        def body(c, cnt):
            off = c * 16
            s = sv[pl.ds(off, 16)]
            gidx = (base + off) + lax.iota(jnp.int32, 16)
            sel = (s > tvv) | ((s == tvv) & (gidx <= jvv))
            seli = sel.astype(jnp.int32)
            pos = cnt + plsc.cumsum(seli) - 1
            plsc.store_scatter(locbuf, [pos], gidx, mask=sel)
            return cnt + jnp.sum(seli)

        cnt = lax.fori_loop(0, _CHUNKS, body, jnp.int32(0))

        npad = lax.rem(jnp.int32(16) - lax.rem(cnt, jnp.int32(16)),
                       jnp.int32(16))
        padmask = lax.iota(jnp.int32, 16) < npad
        plsc.store_scatter(locbuf, [cnt + lax.iota(jnp.int32, 16)],
                           jnp.full((16,), _SENTINEL, jnp.int32), mask=padmask)
        cntp = cnt + npad

        cv[...] = jnp.zeros((16,), jnp.int32) + cntp
        pltpu.sync_copy(cv, cnt_hbm.at[pl.ds(pl.multiple_of(tile * 16, 8), 16)])
        plsc.subcore_barrier()
        pltpu.sync_copy(cnt_hbm.at[pl.ds(pl.multiple_of(cid * 256, 8), 256)], rb)
        off0 = jnp.int32(0)
        tot_c = jnp.int32(0)
        for t in range(16):
            c_t = jnp.max(rb[pl.ds(t * 16, 16)])
            off0 = off0 + jnp.where(sid > t, c_t, 0)
            tot_c = tot_c + c_t
        dest = cid * _HALF + off0

        def wr(k, carry):
            pltpu.sync_copy(locbuf.at[pl.ds(k * 16, 16)],
                            cand_hbm.at[pl.ds(pl.multiple_of(dest + k * 16, 8), 16)])
            return carry

        lax.fori_loop(0, cntp // 16, wr, jnp.int32(0))

        plsc.subcore_barrier()
        for j in range(_HALF // 256):
            gstart = (sid + j * 16) * 16

            @pl.when(gstart < tot_c)
            def _gather(j=j, gstart=gstart):
                gslot = cid * _HALF + gstart
                pltpu.sync_copy(
                    cand_hbm.at[pl.ds(pl.multiple_of(gslot, 8), 16)],
                    idxv.at[pl.ds(j * 16, 16)])
                v = idxv[pl.ds(j * 16, 16)]
                idxv[pl.ds(j * 16, 16)] = jnp.clip(v, 0, _N - 1)
                pltpu.async_copy(
                    img2_hbm.at[idxv.at[pl.ds(j * 16, 16)]],
                    rowsb.at[pl.ds(j * 16, 16)], sem).wait()
                pltpu.sync_copy(
                    rowsb.at[pl.ds(j * 16, 16)],
                    rows_hbm.at[pl.ds(pl.multiple_of(gslot, 8), 16)])

    return compact


_RBLK = 512
_G2 = _SLOTS // _RBLK


def _score2_body(c_ref, r_ref, cnt_ref, t_ref, oc_ref, ov_ref, s_ref):
    i = pl.program_id(0)
    rows = r_ref[...]
    t2 = t_ref[...]
    tn = t2 / jnp.maximum(jnp.sqrt(jnp.sum(t2 * t2)), 1e-8)
    sq = jnp.sum(rows * rows, axis=1, keepdims=True)
    en = rows / jnp.maximum(jnp.sqrt(sq), 1e-8)
    tb = jnp.broadcast_to(tn, (8, _D2)).astype(jnp.bfloat16)
    s = lax.dot_general(en.astype(jnp.bfloat16), tb,
                        (((1,), (1,)), ((), ())),
                        preferred_element_type=jnp.float32)[:, 0:1]
    s_ref[pl.ds(i * (_RBLK // 128), _RBLK // 128), :] = s.reshape(
        _RBLK // 128, 128)

    @pl.when(i == _G2 - 1)
    def _top():
        cand = c_ref[...]
        counts = cnt_ref[...]
        s2 = s_ref[...]

        rows32 = lax.broadcasted_iota(jnp.int32, (32, 16), 0)
        lane16 = lax.broadcasted_iota(jnp.int32, (32, 16), 1)
        cnt_lane0 = jnp.where(lane16 == 0, counts, 0)
        tot0 = jnp.sum(jnp.where(rows32 < 16, cnt_lane0, 0))
        tot1 = jnp.sum(jnp.where(rows32 >= 16, cnt_lane0, 0))

        flat = (lax.broadcasted_iota(jnp.int32, (20, 128), 0) * 128
                + lax.broadcasted_iota(jnp.int32, (20, 128), 1))
        s_in = lax.rem(flat, jnp.int32(_HALF))
        tot = jnp.where(flat < _HALF, tot0, tot1)
        valid = (s_in < tot) & (cand >= 0) & (cand < _N)
        s2 = jnp.where(valid, s2, -jnp.inf)

        big = jnp.int32(2 ** 30)
        out_r = lax.broadcasted_iota(jnp.int32, (8, 128), 0)
        out_l = lax.broadcasted_iota(jnp.int32, (8, 128), 1)
        oc = jnp.zeros((8, 128), jnp.int32)
        ov = jnp.zeros((8, 128), jnp.float32)
        for k in range(_TOPK):
            mx = jnp.max(s2)
            hit = s2 == mx
            fidx = jnp.min(jnp.where(hit, flat, big))
            sel = flat == fidx
            cbest = jnp.sum(jnp.where(sel, cand, 0))
            oc = jnp.where((out_r == 0) & (out_l == k), cbest, oc)
            ov = jnp.where((out_r == 0) & (out_l == k), mx, ov)
            s2 = jnp.where(sel, -jnp.inf, s2)
        oc_ref[...] = oc
        ov_ref[...] = ov


def _top10(cand, rows, counts, text_emb2):
    return pl.pallas_call(
        _score2_body,
        grid=(_G2,),
        in_specs=[
            pl.BlockSpec((20, 128), lambda i: (0, 0)),
            pl.BlockSpec((_RBLK, _D2), lambda i: (i, 0)),
            pl.BlockSpec((32, 16), lambda i: (0, 0)),
            pl.BlockSpec((1, _D2), lambda i: (0, 0)),
        ],
        out_specs=[
            pl.BlockSpec((8, 128), lambda i: (0, 0)),
            pl.BlockSpec((8, 128), lambda i: (0, 0)),
        ],
        out_shape=[
            jax.ShapeDtypeStruct((8, 128), jnp.int32),
            jax.ShapeDtypeStruct((8, 128), jnp.float32),
        ],
        scratch_shapes=[pltpu.VMEM((20, 128), jnp.float32)],
    )(
        cand.reshape(20, 128),
        rows,
        counts.reshape(32, 16),
        text_emb2.reshape(1, _D2),
    )


def kernel(images_emb1, text_emb1, images_emb2, text_emb2, topm, topk):
    scores2d, tj = _stage1(images_emb1, text_emb1)

    tvec = lax.bitcast_convert_type(jnp.broadcast_to(tj[0:1, 0], (16,)),
                                    jnp.float32)
    jvec = jnp.broadcast_to(tj[1:2, 0], (16,))
    cand, counts, rows = _make_compact()(
        scores2d.reshape(_NPAD), images_emb2, tvec, jvec)

    oc, ov = _top10(cand, rows, counts, text_emb2)
    final_candidate = oc[0, :_TOPK]
    top_vals = ov[0, :_TOPK]
    return final_candidate, top_vals


# speedup vs baseline: 1.0750x; 1.0024x over previous
import functools

import jax
import jax.numpy as jnp
import numpy as np
from jax import lax
from jax.experimental import pallas as pl
from jax.experimental.pallas import tpu as pltpu
from jax.experimental.pallas import tpu_sc as plsc

_N = 100000
_D1 = 768
_D2 = 1024
_TOPM = 1000
_TOPK = 10

_BLK = 5120
_G1 = 20
_NPAD = _G1 * _BLK
_MININT = np.int32(-2147483648)
_SENTINEL = np.int32(2 ** 30)

_HALF = 1280
_SLOTS = 2 * _HALF


def _sortable(bits):
    return jnp.where(
        bits < 0,
        jnp.bitwise_xor(jnp.bitwise_not(bits), _MININT),
        bits,
    )


def _stage1_body(x_ref, t_ref, s_ref, tj_ref, k_ref):
    i = pl.program_id(0)
    x = x_ref[...]
    t = t_ref[...]
    tn = t / jnp.maximum(jnp.sqrt(jnp.sum(t * t)), 1e-8)
    sq = jnp.sum(x * x, axis=1, keepdims=True)
    xn = x / jnp.maximum(jnp.sqrt(sq), 1e-8)
    tb = jnp.broadcast_to(tn, (8, _D1)).astype(jnp.bfloat16)
    score = lax.dot_general(xn.astype(jnp.bfloat16), tb,
                            (((1,), (1,)), ((), ())),
                            preferred_element_type=jnp.float32)[:, 0:1]
    gidx = i * _BLK + lax.broadcasted_iota(jnp.int32, (_BLK, 1), 0)
    score = jnp.where(gidx < _N, score, -jnp.inf)
    s_ref[pl.ds(i, 1), :] = score.reshape(1, _BLK)

    @pl.when(i == _G1 - 1)
    def _select():
        bits = lax.bitcast_convert_type(s_ref[...], jnp.int32)
        k_ref[...] = _sortable(bits)

        def t_body(b, t_u):
            cand = jnp.bitwise_or(t_u, lax.shift_left(jnp.int32(1), 31 - b))
            cnt = jnp.sum((k_ref[...] >=
                           jnp.bitwise_xor(cand, _MININT)).astype(jnp.int32))
            return jnp.where(cnt >= _TOPM, cand, t_u)

        t_u = lax.fori_loop(0, 32, t_body, jnp.int32(0))
        t_k = jnp.bitwise_xor(t_u, _MININT)

        flat = (lax.broadcasted_iota(jnp.int32, (_G1, _BLK), 0) * _BLK
                + lax.broadcasted_iota(jnp.int32, (_G1, _BLK), 1))
        n_gt = jnp.sum((k_ref[...] > t_k).astype(jnp.int32))

        def g_of(j):
            eq = k_ref[...] == t_k
            return n_gt + jnp.sum((eq & (flat <= j)).astype(jnp.int32))

        def j_body(b, j_bad):
            cand = jnp.bitwise_or(j_bad, lax.shift_left(jnp.int32(1), 16 - b))
            return jnp.where(g_of(cand) < _TOPM, cand, j_bad)

        j_bad = lax.fori_loop(0, 17, j_body, jnp.int32(0))
        j_fin = jnp.where(g_of(jnp.int32(0)) >= _TOPM, jnp.int32(0), j_bad + 1)

        t_bits = jnp.where(
            t_k >= 0, t_k, jnp.bitwise_not(jnp.bitwise_xor(t_k, _MININT)))
        r = lax.broadcasted_iota(jnp.int32, (8, 128), 0)
        tj_ref[...] = jnp.where(r == 0, t_bits, jnp.where(r == 1, j_fin, 0))


def _stage1(images_emb1, text_emb1):
    return pl.pallas_call(
        _stage1_body,
        grid=(_G1,),
        in_specs=[
            pl.BlockSpec((_BLK, _D1), lambda i: (i, 0)),
            pl.BlockSpec((1, _D1), lambda i: (0, 0)),
        ],
        out_specs=[
            pl.BlockSpec((_G1, _BLK), lambda i: (0, 0)),
            pl.BlockSpec((8, 128), lambda i: (0, 0)),
        ],
        out_shape=[
            jax.ShapeDtypeStruct((_G1, _BLK), jnp.float32),
            jax.ShapeDtypeStruct((8, 128), jnp.int32),
        ],
        scratch_shapes=[pltpu.VMEM((_G1, _BLK), jnp.int32)],
    )(images_emb1, text_emb1.reshape(1, _D1))


_PER_TILE = _NPAD // 32
_CHUNKS = _PER_TILE // 16
_ROWS_PER_TILE = _SLOTS // 32


def _make_compact():
    mesh = plsc.VectorSubcoreMesh(core_axis_name="c", subcore_axis_name="s")

    @functools.partial(
        pl.kernel,
        mesh=mesh,
        compiler_params=pltpu.CompilerParams(needs_layout_passes=False),
        out_type=[
            jax.ShapeDtypeStruct((_SLOTS,), jnp.int32),
            jax.ShapeDtypeStruct((512,), jnp.int32),
            jax.ShapeDtypeStruct((_SLOTS, _D2), jnp.float32),
        ],
        scratch_types=[
            pltpu.VMEM((_PER_TILE,), jnp.float32),
            pltpu.VMEM((16,), jnp.float32),
            pltpu.VMEM((16,), jnp.int32),
            pltpu.VMEM((_PER_TILE + 64,), jnp.int32),
            pltpu.VMEM((16,), jnp.int32),
            pltpu.VMEM((256,), jnp.int32),
            pltpu.VMEM((_ROWS_PER_TILE,), jnp.int32),
            pltpu.VMEM((_ROWS_PER_TILE, _D2), jnp.float32),
            pltpu.SemaphoreType.DMA,
        ],
    )
    def compact(s_hbm, img2_hbm, t_hbm, j_hbm, cand_hbm, cnt_hbm, rows_hbm,
                sv, tv, jv, locbuf, cv, rb, idxv, rowsb, sem):
        cid = lax.axis_index("c")
        sid = lax.axis_index("s")
        tile = cid * 16 + sid
        base = tile * _PER_TILE
        pltpu.sync_copy(s_hbm.at[pl.ds(pl.multiple_of(base, 8), _PER_TILE)], sv)
        pltpu.sync_copy(t_hbm, tv)
        pltpu.sync_copy(j_hbm, jv)
        tvv = tv[...]
        jvv = jv[...]

        def body(c, cnt):
            off = c * 16
            s = sv[pl.ds(off, 16)]
            gidx = (base + off) + lax.iota(jnp.int32, 16)
            sel = (s > tvv) | ((s == tvv) & (gidx <= jvv))
            seli = sel.astype(jnp.int32)
            pos = cnt + plsc.cumsum(seli) - 1
            plsc.store_scatter(locbuf, [pos], gidx, mask=sel)
            return cnt + jnp.sum(seli)

        cnt = lax.fori_loop(0, _CHUNKS, body, jnp.int32(0))

        npad = lax.rem(jnp.int32(16) - lax.rem(cnt, jnp.int32(16)),
                       jnp.int32(16))
        padmask = lax.iota(jnp.int32, 16) < npad
        plsc.store_scatter(locbuf, [cnt + lax.iota(jnp.int32, 16)],
                           jnp.full((16,), _SENTINEL, jnp.int32), mask=padmask)
        cntp = cnt + npad

        cv[...] = jnp.zeros((16,), jnp.int32) + cntp
        pltpu.sync_copy(cv, cnt_hbm.at[pl.ds(pl.multiple_of(tile * 16, 8), 16)])
        plsc.subcore_barrier()
        pltpu.sync_copy(cnt_hbm.at[pl.ds(pl.multiple_of(cid * 256, 8), 256)], rb)
        off0 = jnp.int32(0)
        tot_c = jnp.int32(0)
        for t in range(16):
            c_t = jnp.max(rb[pl.ds(t * 16, 16)])
            off0 = off0 + jnp.where(sid > t, c_t, 0)
            tot_c = tot_c + c_t
        dest = cid * _HALF + off0

        def wr(k, carry):
            pltpu.sync_copy(locbuf.at[pl.ds(k * 16, 16)],
                            cand_hbm.at[pl.ds(pl.multiple_of(dest + k * 16, 8), 16)])
            return carry

        lax.fori_loop(0, cntp // 16, wr, jnp.int32(0))

        plsc.subcore_barrier()
        for j in range(_HALF // 256):
            gstart = (sid + j * 16) * 16

            @pl.when(gstart < tot_c)
            def _gather(j=j, gstart=gstart):
                gslot = cid * _HALF + gstart
                pltpu.sync_copy(
                    cand_hbm.at[pl.ds(pl.multiple_of(gslot, 8), 16)],
                    idxv.at[pl.ds(j * 16, 16)])
                v = idxv[pl.ds(j * 16, 16)]
                idxv[pl.ds(j * 16, 16)] = jnp.clip(v, 0, _N - 1)
                pltpu.async_copy(
                    img2_hbm.at[idxv.at[pl.ds(j * 16, 16)]],
                    rowsb.at[pl.ds(j * 16, 16)], sem).wait()
                pltpu.sync_copy(
                    rowsb.at[pl.ds(j * 16, 16)],
                    rows_hbm.at[pl.ds(pl.multiple_of(gslot, 8), 16)])

    return compact


_RBLK = 512
_G2 = _SLOTS // _RBLK


def _score2_body(c_ref, r_ref, cnt_ref, t_ref, oc_ref, ov_ref, s_ref):
    i = pl.program_id(0)
    rows = r_ref[...]
    t2 = t_ref[...]
    tn = t2 / jnp.maximum(jnp.sqrt(jnp.sum(t2 * t2)), 1e-8)
    sq = jnp.sum(rows * rows, axis=1, keepdims=True)
    en = rows / jnp.maximum(jnp.sqrt(sq), 1e-8)
    tb = jnp.broadcast_to(tn, (8, _D2)).astype(jnp.bfloat16)
    s = lax.dot_general(en.astype(jnp.bfloat16), tb,
                        (((1,), (1,)), ((), ())),
                        preferred_element_type=jnp.float32)[:, 0:1]
    s_ref[pl.ds(i * (_RBLK // 128), _RBLK // 128), :] = s.reshape(
        _RBLK // 128, 128)

    @pl.when(i == _G2 - 1)
    def _top():
        cand = c_ref[...]
        counts = cnt_ref[...]
        s2 = s_ref[...]

        rows32 = lax.broadcasted_iota(jnp.int32, (32, 16), 0)
        lane16 = lax.broadcasted_iota(jnp.int32, (32, 16), 1)
        cnt_lane0 = jnp.where(lane16 == 0, counts, 0)
        tot0 = jnp.sum(jnp.where(rows32 < 16, cnt_lane0, 0))
        tot1 = jnp.sum(jnp.where(rows32 >= 16, cnt_lane0, 0))

        flat = (lax.broadcasted_iota(jnp.int32, (20, 128), 0) * 128
                + lax.broadcasted_iota(jnp.int32, (20, 128), 1))
        s_in = lax.rem(flat, jnp.int32(_HALF))
        tot = jnp.where(flat < _HALF, tot0, tot1)
        valid = (s_in < tot) & (cand >= 0) & (cand < _N)
        s2 = jnp.where(valid, s2, -jnp.inf)

        big = jnp.int32(2 ** 30)
        out_r = lax.broadcasted_iota(jnp.int32, (8, 128), 0)
        out_l = lax.broadcasted_iota(jnp.int32, (8, 128), 1)
        oc = jnp.zeros((8, 128), jnp.int32)
        ov = jnp.zeros((8, 128), jnp.float32)
        for k in range(_TOPK):
            mx = jnp.max(s2)
            hit = s2 == mx
            fidx = jnp.min(jnp.where(hit, flat, big))
            sel = flat == fidx
            cbest = jnp.sum(jnp.where(sel, cand, 0))
            oc = jnp.where((out_r == 0) & (out_l == k), cbest, oc)
            ov = jnp.where((out_r == 0) & (out_l == k), mx, ov)
            s2 = jnp.where(sel, -jnp.inf, s2)
        oc_ref[...] = oc
        ov_ref[...] = ov


def _top10(cand, rows, counts, text_emb2):
    return pl.pallas_call(
        _score2_body,
        grid=(_G2,),
        in_specs=[
            pl.BlockSpec((20, 128), lambda i: (0, 0)),
            pl.BlockSpec((_RBLK, _D2), lambda i: (i, 0)),
            pl.BlockSpec((32, 16), lambda i: (0, 0)),
            pl.BlockSpec((1, _D2), lambda i: (0, 0)),
        ],
        out_specs=[
            pl.BlockSpec((8, 128), lambda i: (0, 0)),
            pl.BlockSpec((8, 128), lambda i: (0, 0)),
        ],
        out_shape=[
            jax.ShapeDtypeStruct((8, 128), jnp.int32),
            jax.ShapeDtypeStruct((8, 128), jnp.float32),
        ],
        scratch_shapes=[pltpu.VMEM((20, 128), jnp.float32)],
    )(
        cand.reshape(20, 128),
        rows,
        counts.reshape(32, 16),
        text_emb2.reshape(1, _D2),
    )


def kernel(images_emb1, text_emb1, images_emb2, text_emb2, topm, topk):
    scores2d, tj = _stage1(images_emb1, text_emb1)

    tvec = lax.bitcast_convert_type(jnp.broadcast_to(tj[0:1, 0], (16,)),
                                    jnp.float32)
    jvec = jnp.broadcast_to(tj[1:2, 0], (16,))
    cand, counts, rows = _make_compact()(
        scores2d.reshape(_NPAD), images_emb2, tvec, jvec)

    oc, ov = _top10(cand, rows, counts, text_emb2)
    final_candidate = oc[0, :_TOPK]
    top_vals = ov[0, :_TOPK]
    return final_candidate, top_vals
